# baseline (speedup 1.0000x reference)
import math

import jax
import jax.numpy as jnp
from jax.experimental import pallas as pl
from jax.experimental.pallas import tpu as pltpu

_C = 100000
_SMOOTHING = 0.1
_CONF = 1.0 - _SMOOTHING
_EPS = _SMOOTHING / (_C - 1)

_R = 8
_K = 8192


def _loss_kernel(tgt_ref, pred_ref, out_ref, m_ref, s_ref, sp_ref, ts_ref):
    rb = pl.program_id(0)
    kc = pl.program_id(1)
    nk = pl.num_programs(1)

    x = pred_ref[...]
    col = jax.lax.broadcasted_iota(jnp.int32, (_R, _K), 1) + kc * _K
    valid = col < _C
    xm = jnp.where(valid, x, -jnp.inf)
    cmax = jnp.max(xm, axis=1, keepdims=True)
    csum = jnp.sum(jnp.where(valid, x, 0.0), axis=1, keepdims=True)
    tgt = tgt_ref[0, 0, :].reshape(_R, 1)
    tsel = jnp.sum(jnp.where(col == tgt, x, 0.0), axis=1, keepdims=True)

    @pl.when(kc == 0)
    def _init():
        m_ref[...] = cmax
        s_ref[...] = jnp.sum(jnp.exp(xm - cmax), axis=1, keepdims=True)
        sp_ref[...] = csum
        ts_ref[...] = tsel

    @pl.when(kc > 0)
    def _update():
        m_old = m_ref[...]
        m_new = jnp.maximum(m_old, cmax)
        s_ref[...] = s_ref[...] * jnp.exp(m_old - m_new) + jnp.sum(
            jnp.exp(xm - m_new), axis=1, keepdims=True)
        m_ref[...] = m_new
        sp_ref[...] = sp_ref[...] + csum
        ts_ref[...] = ts_ref[...] + tsel

    @pl.when(kc == nk - 1)
    def _finalize():
        lse = m_ref[...] + jnp.log(s_ref[...])
        rowsum_logp = sp_ref[...] - _C * lse
        logp_t = ts_ref[...] - lse
        contrib = -(_EPS * rowsum_logp + (_CONF - _EPS) * logp_t)
        val = jnp.sum(contrib)

        @pl.when(rb == 0)
        def _():
            out_ref[0, 0] = val

        @pl.when(rb > 0)
        def _():
            out_ref[0, 0] = out_ref[0, 0] + val


@jax.jit
def kernel(pred, target):
    B = pred.shape[0]
    nb = B // _R
    nk = pl.cdiv(_C, _K)
    tgt3 = target.astype(jnp.int32).reshape(nb, 1, _R)

    acc = pl.pallas_call(
        _loss_kernel,
        grid=(nb, nk),
        in_specs=[
            pl.BlockSpec((1, 1, _R), lambda rb, kc: (rb, 0, 0)),
            pl.BlockSpec((_R, _K), lambda rb, kc: (rb, kc)),
        ],
        out_specs=pl.BlockSpec(
            (1, 1), lambda rb, kc: (0, 0), memory_space=pltpu.SMEM),
        out_shape=jax.ShapeDtypeStruct((1, 1), jnp.float32),
        scratch_shapes=[
            pltpu.VMEM((_R, 1), jnp.float32),
            pltpu.VMEM((_R, 1), jnp.float32),
            pltpu.VMEM((_R, 1), jnp.float32),
            pltpu.VMEM((_R, 1), jnp.float32),
        ],
    )(tgt3, pred)

    k0 = (_C - 1) * _EPS * math.log(_EPS) + _CONF * math.log(_CONF)
    return (acc[0, 0] + B * k0) / (B * _C)
